# baseline (speedup 1.0000x reference)
import functools

import jax
import jax.numpy as jnp
from jax import lax
from jax.experimental import pallas as pl
from jax.experimental.pallas import tpu as pltpu
from jax.experimental.pallas import tpu_sc as plsc

NUM_WORKERS = 32
LANES = 16


def _build_kernel(B, T, D):
    rows_per_w = (B * T) // NUM_WORKERS
    batches_per_w = B // NUM_WORKERS
    mesh = plsc.VectorSubcoreMesh(core_axis_name="c", subcore_axis_name="s")

    @functools.partial(
        pl.kernel,
        mesh=mesh,
        compiler_params=pltpu.CompilerParams(use_tc_tiling_on_sc=False),
        out_type=jax.ShapeDtypeStruct((B * T, D), jnp.float32),
        scratch_types=[
            pltpu.VMEM((rows_per_w,), jnp.int32),
            pltpu.VMEM((T, D), jnp.float32),
            pltpu.VMEM((T, D), jnp.float32),
            pltpu.SemaphoreType.DMA,
        ],
    )
    def emb_kernel(idx_hbm, table_hbm, pos_hbm, out_hbm, idx_v, pos_v, rows_v, sem):
        wid = lax.axis_index("s") * 2 + lax.axis_index("c")
        base = wid * rows_per_w
        pltpu.sync_copy(pos_hbm, pos_v)
        pltpu.sync_copy(idx_hbm.at[pl.ds(base, rows_per_w)], idx_v)

        def batch_body(b, carry):
            r0 = b * T
            cp1 = pltpu.async_copy(
                table_hbm.at[idx_v.at[pl.ds(r0, 128)]],
                rows_v.at[pl.ds(0, 128)],
                sem,
            )
            cp2 = pltpu.async_copy(
                table_hbm.at[idx_v.at[pl.ds(r0 + 128, T - 128)]],
                rows_v.at[pl.ds(128, T - 128)],
                sem,
            )
            cp1.wait()
            cp2.wait()

            def row_body(t, c2):
                for ci in range(D // LANES):
                    s = pl.ds(ci * LANES, LANES)
                    rows_v[t, s] = rows_v[t, s] + pos_v[t, s]
                return c2

            lax.fori_loop(0, T, row_body, 0)
            pltpu.sync_copy(rows_v, out_hbm.at[pl.ds(base + r0, T)])
            return carry

        lax.fori_loop(0, batches_per_w, batch_body, 0)

    return emb_kernel


def kernel(x, token_table, pos_table):
    B, T = x.shape
    V, D = token_table.shape
    flat_idx = x.reshape(B * T).astype(jnp.int32)
    out = _build_kernel(B, T, D)(flat_idx, token_table, pos_table)
    return out.reshape(B, T, D)
